# baseline (speedup 1.0000x reference)
import jax
import jax.numpy as jnp
from jax import lax
from jax.experimental import pallas as pl
from jax.experimental.pallas import tpu as pltpu
from jax.experimental.pallas import tpu_sc as plsc

B = 16384
D = 128
L = 16
NC = 2
NS = 16
NW = NC * NS
B_PER_W = B // NW
CHUNK = 128
NCHUNK = B_PER_W // CHUNK
GROUPS = CHUNK // L
NBUF = 3


def _body(uidx_hbm, iidx_hbm, utab_hbm, itab_hbm, out_hbm, *scr):
  uidx_v, iidx_v = scr[0], scr[1]
  ubufs = list(scr[2:2 + NBUF])
  ibufs = list(scr[2 + NBUF:2 + 2 * NBUF])
  part, part2, out_v = scr[2 + 2 * NBUF], scr[3 + 2 * NBUF], scr[4 + 2 * NBUF]
  sus = list(scr[5 + 2 * NBUF:5 + 3 * NBUF])
  sis = list(scr[5 + 3 * NBUF:5 + 4 * NBUF])
  sx = scr[5 + 4 * NBUF]

  wid = lax.axis_index("s") * NC + lax.axis_index("c")
  base = wid * B_PER_W

  stage = []
  for c in range(NCHUNK):
    stage.append((
        pltpu.async_copy(
            uidx_hbm.at[pl.ds(base + c * CHUNK, CHUNK)], uidx_v.at[c], sx),
        pltpu.async_copy(
            iidx_hbm.at[pl.ds(base + c * CHUNK, CHUNK)], iidx_v.at[c], sx),
    ))

  iot = lax.iota(jnp.int32, L)

  def start(c):
    p = c % NBUF
    stage[c][0].wait()
    stage[c][1].wait()
    cu = pltpu.async_copy(utab_hbm.at[uidx_v.at[c]], ubufs[p], sus[p])
    ci = pltpu.async_copy(itab_hbm.at[iidx_v.at[c]], ibufs[p], sis[p])
    return cu, ci

  pend = {}
  for c in range(min(NBUF, NCHUNK)):
    pend[c] = start(c)

  outcp = []

  for c in range(NCHUNK):
    p = c % NBUF
    pend[c][0].wait()
    pend[c][1].wait()
    ur, ir = ubufs[p], ibufs[p]

    def group_body(g2, carry, ur=ur, ir=ir, c=c):
      for half, pt in ((0, part), (1, part2)):
        g = g2 * 2 + half
        for r in range(L):
          row = g * L + r
          acc = ur[row, 0:L] * ir[row, 0:L]
          for k in range(1, D // L):
            acc = acc + ur[row, k * L:(k + 1) * L] * ir[row, k * L:(k + 1) * L]
          pt[pl.ds(r * (L + 1), L)] = acc
        res = plsc.load_gather(pt, [iot * (L + 1)])
        for cc in range(1, L):
          res = res + plsc.load_gather(pt, [iot * (L + 1) + cc])
        out_v[pl.ds(c * CHUNK + g * L, L)] = res
      return carry

    lax.fori_loop(0, GROUPS // 2, group_body, 0)
    if c + NBUF < NCHUNK:
      pend[c + NBUF] = start(c + NBUF)
    outcp.append(pltpu.async_copy(
        out_v.at[pl.ds(c * CHUNK, CHUNK)],
        out_hbm.at[pl.ds(base + c * CHUNK, CHUNK)], sx))

  for cp in outcp:
    cp.wait()


@jax.jit
def _run(user_indices, item_indices, user_table, item_table):
  mesh = plsc.VectorSubcoreMesh(core_axis_name="c", subcore_axis_name="s")
  f = pl.kernel(
      _body,
      out_type=jax.ShapeDtypeStruct((B,), jnp.float32),
      mesh=mesh,
      compiler_params=pltpu.CompilerParams(needs_layout_passes=False),
      scratch_types=(
          [pltpu.VMEM((NCHUNK, CHUNK), jnp.int32)] * 2
          + [pltpu.VMEM((CHUNK, D), jnp.float32)] * (2 * NBUF)
          + [pltpu.VMEM((L * (L + 1),), jnp.float32),
             pltpu.VMEM((L * (L + 1),), jnp.float32),
             pltpu.VMEM((B_PER_W,), jnp.float32)]
          + [pltpu.SemaphoreType.DMA] * (2 * NBUF + 1)
      ),
  )
  return f(user_indices, item_indices, user_table, item_table)


def kernel(user_indices, item_indices, user_table, item_table):
  return _run(user_indices.astype(jnp.int32), item_indices.astype(jnp.int32),
              user_table, item_table)


# speedup vs baseline: 1.1148x; 1.1148x over previous
import jax
import jax.numpy as jnp
from jax import lax
from jax.experimental import pallas as pl
from jax.experimental.pallas import tpu as pltpu
from jax.experimental.pallas import tpu_sc as plsc

B = 16384
D = 128
L = 16
NC = 2
NS = 16
NW = NC * NS
B_PER_W = B // NW
CHUNK = 128
NCHUNK = B_PER_W // CHUNK
GROUPS = CHUNK // L
NBUF = 3


def _body(uidx_hbm, iidx_hbm, utab_hbm, itab_hbm, out_hbm, *scr):
  uidx_v, iidx_v = scr[0], scr[1]
  ubufs = list(scr[2:2 + NBUF])
  ibufs = list(scr[2 + NBUF:2 + 2 * NBUF])
  part, out_v = scr[2 + 2 * NBUF], scr[3 + 2 * NBUF]
  sus = list(scr[4 + 2 * NBUF:4 + 3 * NBUF])
  sis = list(scr[4 + 3 * NBUF:4 + 4 * NBUF])
  sx = scr[4 + 4 * NBUF]

  wid = lax.axis_index("s") * NC + lax.axis_index("c")
  base = wid * B_PER_W

  stage = []
  for c in range(NCHUNK):
    stage.append((
        pltpu.async_copy(
            uidx_hbm.at[pl.ds(base + c * CHUNK, CHUNK)], uidx_v.at[c], sx),
        pltpu.async_copy(
            iidx_hbm.at[pl.ds(base + c * CHUNK, CHUNK)], iidx_v.at[c], sx),
    ))

  iot = lax.iota(jnp.int32, L)

  def start(c):
    p = c % NBUF
    stage[c][0].wait()
    stage[c][1].wait()
    cu = pltpu.async_copy(utab_hbm.at[uidx_v.at[c]], ubufs[p], sus[p])
    ci = pltpu.async_copy(itab_hbm.at[iidx_v.at[c]], ibufs[p], sis[p])
    return cu, ci

  pend = {}
  for c in range(min(NBUF, NCHUNK)):
    pend[c] = start(c)

  outcp = []

  for c in range(NCHUNK):
    p = c % NBUF
    pend[c][0].wait()
    pend[c][1].wait()
    ur, ir = ubufs[p], ibufs[p]

    def group_body(g, carry, ur=ur, ir=ir, c=c):
      for r in range(L):
        row = g * L + r
        acc = ur[row, 0:L] * ir[row, 0:L]
        for k in range(1, D // L):
          acc = acc + ur[row, k * L:(k + 1) * L] * ir[row, k * L:(k + 1) * L]
        part[pl.ds(r * (L + 1), L)] = acc
      res = plsc.load_gather(part, [iot * (L + 1)])
      for cc in range(1, L):
        res = res + plsc.load_gather(part, [iot * (L + 1) + cc])
      out_v[pl.ds(c * CHUNK + g * L, L)] = res
      return carry

    lax.fori_loop(0, GROUPS, group_body, 0)
    if c + NBUF < NCHUNK:
      pend[c + NBUF] = start(c + NBUF)
    outcp.append(pltpu.async_copy(
        out_v.at[pl.ds(c * CHUNK, CHUNK)],
        out_hbm.at[pl.ds(base + c * CHUNK, CHUNK)], sx))

  for cp in outcp:
    cp.wait()


@jax.jit
def _run(user_indices, item_indices, user_table, item_table):
  mesh = plsc.VectorSubcoreMesh(core_axis_name="c", subcore_axis_name="s")
  f = pl.kernel(
      _body,
      out_type=jax.ShapeDtypeStruct((B,), jnp.float32),
      mesh=mesh,
      compiler_params=pltpu.CompilerParams(needs_layout_passes=False),
      scratch_types=(
          [pltpu.VMEM((NCHUNK, CHUNK), jnp.int32)] * 2
          + [pltpu.VMEM((CHUNK, D), jnp.float32)] * (2 * NBUF)
          + [pltpu.VMEM((L * (L + 1),), jnp.float32),
             pltpu.VMEM((B_PER_W,), jnp.float32)]
          + [pltpu.SemaphoreType.DMA] * (2 * NBUF + 1)
      ),
  )
  return f(user_indices, item_indices, user_table, item_table)


def kernel(user_indices, item_indices, user_table, item_table):
  return _run(user_indices.astype(jnp.int32), item_indices.astype(jnp.int32),
              user_table, item_table)


# speedup vs baseline: 1.1242x; 1.0084x over previous
import jax
import jax.numpy as jnp
from jax import lax
from jax.experimental import pallas as pl
from jax.experimental.pallas import tpu as pltpu
from jax.experimental.pallas import tpu_sc as plsc

B = 16384
D = 128
L = 16
NC = 2
NS = 16
NW = NC * NS
B_PER_W = B // NW
CHUNK = 128
NCHUNK = B_PER_W // CHUNK
GROUPS = CHUNK // L
NBUF = 3


def _body(uidx_hbm, iidx_hbm, utab_hbm, itab_hbm, out_hbm, *scr):
  uidx_v, iidx_v = scr[0], scr[1]
  ubufs = list(scr[2:2 + NBUF])
  ibufs = list(scr[2 + NBUF:2 + NBUF + NCHUNK])
  k = 2 + NBUF + NCHUNK
  part, out_v = scr[k], scr[k + 1]
  sus = list(scr[k + 2:k + 2 + NBUF])
  sis = list(scr[k + 2 + NBUF:k + 2 + NBUF + NCHUNK])
  sx = scr[k + 2 + NBUF + NCHUNK]

  wid = lax.axis_index("s") * NC + lax.axis_index("c")
  base = wid * B_PER_W

  stage = []
  for c in range(NCHUNK):
    stage.append((
        pltpu.async_copy(
            uidx_hbm.at[pl.ds(base + c * CHUNK, CHUNK)], uidx_v.at[c], sx),
        pltpu.async_copy(
            iidx_hbm.at[pl.ds(base + c * CHUNK, CHUNK)], iidx_v.at[c], sx),
    ))

  iot = lax.iota(jnp.int32, L)

  def start_u(c):
    p = c % NBUF
    stage[c][0].wait()
    return pltpu.async_copy(utab_hbm.at[uidx_v.at[c]], ubufs[p], sus[p])

  def start_i(c):
    stage[c][1].wait()
    return pltpu.async_copy(itab_hbm.at[iidx_v.at[c]], ibufs[c], sis[c])

  pend = {}
  for c in range(min(NBUF, NCHUNK)):
    pend[c] = start_u(c)
  pend_i = [start_i(c) for c in range(NCHUNK)]

  outcp = []

  for c in range(NCHUNK):
    p = c % NBUF
    pend[c].wait()
    pend_i[c].wait()
    ur, ir = ubufs[p], ibufs[c]

    def group_body(g, carry, ur=ur, ir=ir, c=c):
      for r in range(L):
        row = g * L + r
        acc = ur[row, 0:L] * ir[row, 0:L]
        for k in range(1, D // L):
          acc = acc + ur[row, k * L:(k + 1) * L] * ir[row, k * L:(k + 1) * L]
        part[pl.ds(r * (L + 1), L)] = acc
      res = plsc.load_gather(part, [iot * (L + 1)])
      for cc in range(1, L):
        res = res + plsc.load_gather(part, [iot * (L + 1) + cc])
      out_v[pl.ds(c * CHUNK + g * L, L)] = res
      return carry

    lax.fori_loop(0, GROUPS, group_body, 0)
    if c + NBUF < NCHUNK:
      pend[c + NBUF] = start_u(c + NBUF)
    outcp.append(pltpu.async_copy(
        out_v.at[pl.ds(c * CHUNK, CHUNK)],
        out_hbm.at[pl.ds(base + c * CHUNK, CHUNK)], sx))

  for cp in outcp:
    cp.wait()


@jax.jit
def _run(user_indices, item_indices, user_table, item_table):
  mesh = plsc.VectorSubcoreMesh(core_axis_name="c", subcore_axis_name="s")
  f = pl.kernel(
      _body,
      out_type=jax.ShapeDtypeStruct((B,), jnp.float32),
      mesh=mesh,
      compiler_params=pltpu.CompilerParams(needs_layout_passes=False),
      scratch_types=(
          [pltpu.VMEM((NCHUNK, CHUNK), jnp.int32)] * 2
          + [pltpu.VMEM((CHUNK, D), jnp.float32)] * (NBUF + NCHUNK)
          + [pltpu.VMEM((L * (L + 1),), jnp.float32),
             pltpu.VMEM((B_PER_W,), jnp.float32)]
          + [pltpu.SemaphoreType.DMA] * (NBUF + NCHUNK + 1)
      ),
  )
  return f(user_indices, item_indices, user_table, item_table)


def kernel(user_indices, item_indices, user_table, item_table):
  return _run(user_indices.astype(jnp.int32), item_indices.astype(jnp.int32),
              user_table, item_table)
